# Initial kernel scaffold; baseline (speedup 1.0000x reference)
import functools
import math

import jax
import jax.numpy as jnp
from jax import lax
from jax.experimental import pallas as pl
from jax.experimental.pallas import tpu as pltpu
from jax.experimental.pallas import tpu_sc as plsc

HIDDEN = 256
HEADS = 16
DH = 16
HALF = HIDDEN // 2
HPC = HEADS // 2
INV_SCALE = 1.0 / math.sqrt(DH)

NC = 2
NS = 16
CHUNK = 128
ROWS = 8


def _sc_attention(n_nodes, n_pad, e_pad, kt, qt, vt, src_pad, dst_pad):
    ept = e_pad // NS
    nchunks = ept // CHUNK
    npt = n_pad // NS
    mesh = plsc.VectorSubcoreMesh(core_axis_name="c", subcore_axis_name="s")

    @functools.partial(
        pl.kernel,
        out_type=jax.ShapeDtypeStruct((NC * n_pad, HALF), jnp.float32),
        mesh=mesh,
        scratch_types=[
            pltpu.VMEM((CHUNK,), jnp.int32),
            pltpu.VMEM((CHUNK,), jnp.int32),
            pltpu.VMEM((CHUNK,), jnp.int32),
            pltpu.VMEM((CHUNK, HALF), jnp.float32),
            pltpu.VMEM((CHUNK, HALF), jnp.float32),
            pltpu.VMEM((CHUNK, HALF), jnp.float32),
            pltpu.VMEM((CHUNK, HALF), jnp.float32),
            pltpu.VMEM((CHUNK, DH), jnp.float32),
            pltpu.VMEM((ROWS, HALF), jnp.float32),
            pltpu.VMEM((ROWS, DH), jnp.float32),
            pltpu.MemorySpace.VMEM_SHARED((n_pad, HALF), jnp.float32),
            pltpu.MemorySpace.VMEM_SHARED((n_pad, DH), jnp.float32),
            pltpu.SemaphoreType.DMA,
        ],
    )
    def attn(kt_hbm, qt_hbm, vt_hbm, src_hbm, dst_hbm, out_hbm,
             sidx, didx, didx_g, krows, qrows, vrows, msg, srow,
             wvb, zb, wv_sh, z_sh, sem):
        c = lax.axis_index("c")
        s = lax.axis_index("s")
        cbase = c * n_nodes
        lane = lax.iota(jnp.int32, DH)
        zeros16 = jnp.zeros((DH,), jnp.float32)

        for i in range(ROWS):
            for h in range(HPC):
                wvb[i, pl.ds(h * DH, DH)] = zeros16
            zb[i, :] = zeros16

        @pl.loop(0, npt // ROWS)
        def _zero(r):
            row0 = s * npt + r * ROWS
            pltpu.sync_copy(wvb, wv_sh.at[pl.ds(row0, ROWS)])
            pltpu.sync_copy(zb, z_sh.at[pl.ds(row0, ROWS)])

        plsc.subcore_barrier()

        @pl.loop(0, nchunks)
        def _chunk(ch):
            base = s * ept + ch * CHUNK
            pltpu.sync_copy(src_hbm.at[pl.ds(base, CHUNK)], sidx)
            pltpu.sync_copy(dst_hbm.at[pl.ds(base, CHUNK)], didx)
            for j in range(CHUNK // DH):
                sl = pl.ds(j * DH, DH)
                sidx[sl] = sidx[sl] + cbase
                didx_g[sl] = didx[sl] + cbase
            d1 = pltpu.async_copy(kt_hbm.at[sidx], krows, sem)
            d2 = pltpu.async_copy(qt_hbm.at[didx_g], qrows, sem)
            d3 = pltpu.async_copy(vt_hbm.at[sidx], vrows, sem)
            d1.wait()
            d2.wait()
            d3.wait()

            @pl.loop(0, CHUNK)
            def _edge(e):
                sv = zeros16
                for h in range(HPC):
                    sl = pl.ds(h * DH, DH)
                    dot = jnp.sum(krows[e, sl] * qrows[e, sl])
                    sv = sv + jnp.where(lane == h, dot, 0.0)
                sv = jnp.clip(sv * INV_SCALE, -5.0, 5.0)
                sv = jnp.where(lane < HPC, jnp.exp(sv), 0.0)
                srow[e, :] = sv
                for h in range(HPC):
                    sl = pl.ds(h * DH, DH)
                    msg[e, sl] = vrows[e, sl] * sv[h]

            pltpu.sync_copy(msg, wv_sh.at[didx], add=True)
            pltpu.sync_copy(srow, z_sh.at[didx], add=True)

        plsc.subcore_barrier()

        @pl.loop(0, npt // ROWS)
        def _norm(r):
            row0 = s * npt + r * ROWS
            pltpu.sync_copy(wv_sh.at[pl.ds(row0, ROWS)], wvb)
            pltpu.sync_copy(z_sh.at[pl.ds(row0, ROWS)], zb)
            for i in range(ROWS):
                zrow = zb[i, :] + 1e-6
                for h in range(HPC):
                    sl = pl.ds(h * DH, DH)
                    wvb[i, sl] = wvb[i, sl] / zrow[h]
            pltpu.sync_copy(wvb, out_hbm.at[pl.ds(c * n_pad + row0, ROWS)])

    return attn(kt, qt, vt, src_pad, dst_pad)


def kernel(q, k, v, edge_index):
    b, n, _ = q.shape
    n_pad = ((n + NS * ROWS - 1) // (NS * ROWS)) * (NS * ROWS)
    e = edge_index.shape[1]
    e_pad = ((e + NS * CHUNK - 1) // (NS * CHUNK)) * (NS * CHUNK)

    def halves(x):
        return x.reshape(n, 2, HALF).transpose(1, 0, 2).reshape(2 * n, HALF)

    kt = halves(k.reshape(n, HIDDEN))
    qt = halves(q.reshape(n, HIDDEN))
    vt = halves(v.reshape(n, HIDDEN))

    src = edge_index[0].astype(jnp.int32)
    dst = edge_index[1].astype(jnp.int32)
    pad = e_pad - e
    src_pad = jnp.concatenate([src, jnp.zeros((pad,), jnp.int32)])
    dst_pad = jnp.concatenate([dst, jnp.full((pad,), n, jnp.int32)])

    out = _sc_attention(n, n_pad, e_pad, kt, qt, vt, src_pad, dst_pad)
    x = jnp.concatenate([out[:n], out[n_pad:n_pad + n]], axis=1)
    return x.reshape(b, n, HIDDEN)


# speedup vs baseline: 19.4044x; 19.4044x over previous
import functools
import math

import jax
import jax.numpy as jnp
from jax import lax
from jax.experimental import pallas as pl
from jax.experimental.pallas import tpu as pltpu
from jax.experimental.pallas import tpu_sc as plsc

HIDDEN = 256
HEADS = 16
DH = 16
HALF = HIDDEN // 2
HPC = HEADS // 2
INV_SCALE = 1.0 / math.sqrt(DH)

NC = 2
NS = 16
CHUNK = 64
NR = 16


def _sc_attention(n_nodes, n_pad, e_pad, kt, qt, vt, src_pad, dst_pad):
    ept = e_pad // NS
    nchunks = ept // CHUNK
    npt = n_pad // NS
    mesh = plsc.VectorSubcoreMesh(core_axis_name="c", subcore_axis_name="s")

    @functools.partial(
        pl.kernel,
        out_type=(
            jax.ShapeDtypeStruct((NC * n_pad, HALF), jnp.float32),
            jax.ShapeDtypeStruct((NC * e_pad, DH), jnp.float32),
        ),
        mesh=mesh,
        compiler_params=pltpu.CompilerParams(needs_layout_passes=False),
        scratch_types=[
            pltpu.VMEM((CHUNK,), jnp.int32),
            pltpu.VMEM((CHUNK,), jnp.int32),
            pltpu.VMEM((CHUNK,), jnp.int32),
            pltpu.VMEM((CHUNK, HALF), jnp.float32),
            pltpu.VMEM((CHUNK, HALF), jnp.float32),
            pltpu.VMEM((CHUNK, HALF), jnp.float32),
            pltpu.VMEM((CHUNK, DH), jnp.float32),
            pltpu.VMEM((NR,), jnp.int32),
            pltpu.VMEM((NR, HALF), jnp.float32),
            pltpu.VMEM((NR, HALF), jnp.float32),
            pltpu.MemorySpace.VMEM_SHARED((n_pad, HALF), jnp.float32),
            pltpu.SemaphoreType.DMA,
        ],
    )
    def attn(kt_hbm, qt_hbm, vt_hbm, src_hbm, dst_hbm, out_hbm, sco_hbm,
             sidx, didx, didx_g, krows, qrows, vrows, srow,
             oidx, wvb, zb, acc_sh, sem):
        c = lax.axis_index("c")
        s = lax.axis_index("s")
        cbase = c * (n_nodes + 1)
        lane = lax.iota(jnp.int32, DH)
        zeros16 = jnp.zeros((DH,), jnp.float32)

        for i in range(NR):
            for h in range(HPC):
                zb[i, pl.ds(h * DH, DH)] = zeros16

        def zero_acc():
            @pl.loop(0, npt // NR)
            def _zero(r):
                row0 = s * npt + r * NR
                oidx[:] = row0 + lane
                pltpu.async_copy(zb, acc_sh.at[oidx], sem).wait()

        zero_acc()
        plsc.subcore_barrier()

        @pl.loop(0, nchunks)
        def _chunk_a(ch):
            base = s * ept + ch * CHUNK
            pltpu.sync_copy(src_hbm.at[pl.ds(base, CHUNK)], sidx)
            pltpu.sync_copy(dst_hbm.at[pl.ds(base, CHUNK)], didx)
            for j in range(CHUNK // DH):
                sl = pl.ds(j * DH, DH)
                sidx[sl] = sidx[sl] + cbase
                didx_g[sl] = didx[sl] + cbase
            d1 = pltpu.async_copy(kt_hbm.at[sidx], krows, sem)
            d2 = pltpu.async_copy(qt_hbm.at[didx_g], qrows, sem)
            d3 = pltpu.async_copy(vt_hbm.at[sidx], vrows, sem)
            d1.wait()
            d2.wait()
            d3.wait()

            @pl.loop(0, CHUNK)
            def _edge(e):
                sv = zeros16
                for h in range(HPC):
                    sl = pl.ds(h * DH, DH)
                    dot = jnp.sum(krows[e, sl] * qrows[e, sl])
                    sv = sv + jnp.where(lane == h, dot, 0.0)
                sv = jnp.clip(sv * INV_SCALE, -5.0, 5.0)
                sv = jnp.exp(sv)
                srow[e, :] = sv
                for h in range(HPC):
                    sl = pl.ds(h * DH, DH)
                    vrows[e, sl] = vrows[e, sl] * sv[h]

            d4 = pltpu.async_copy(vrows, acc_sh.at[didx], sem, add=True)
            d4.wait()
            pltpu.sync_copy(srow, sco_hbm.at[pl.ds(c * e_pad + base, CHUNK)])

        plsc.subcore_barrier()

        @pl.loop(0, npt // NR)
        def _stage(r):
            row0 = s * npt + r * NR
            oidx[:] = row0 + lane
            pltpu.async_copy(acc_sh.at[oidx], wvb, sem).wait()
            pltpu.sync_copy(wvb, out_hbm.at[pl.ds(c * n_pad + row0, NR)])

        zero_acc()
        plsc.subcore_barrier()

        @pl.loop(0, nchunks)
        def _chunk_b(ch):
            base = s * ept + ch * CHUNK
            pltpu.sync_copy(dst_hbm.at[pl.ds(base, CHUNK)], didx)
            pltpu.sync_copy(sco_hbm.at[pl.ds(c * e_pad + base, CHUNK)], srow)

            @pl.loop(0, CHUNK)
            def _edge_b(e):
                sv = srow[e, :]
                for h in range(HPC):
                    vrows[e, pl.ds(h * DH, DH)] = sv[h] + zeros16

            d1 = pltpu.async_copy(vrows, acc_sh.at[didx], sem, add=True)
            d1.wait()

        plsc.subcore_barrier()

        @pl.loop(0, npt // NR)
        def _norm(r):
            row0 = s * npt + r * NR
            oidx[:] = row0 + lane
            d1 = pltpu.async_copy(acc_sh.at[oidx], zb, sem)
            pltpu.sync_copy(out_hbm.at[pl.ds(c * n_pad + row0, NR)], wvb)
            d1.wait()
            for i in range(NR):
                for h in range(HPC):
                    sl = pl.ds(h * DH, DH)
                    wvb[i, sl] = wvb[i, sl] / (zb[i, sl] + 1e-6)
            pltpu.sync_copy(wvb, out_hbm.at[pl.ds(c * n_pad + row0, NR)])

    return attn(kt, qt, vt, src_pad, dst_pad)


def kernel(q, k, v, edge_index):
    b, n, _ = q.shape
    n_pad = ((n + 1 + NR * NS - 1) // (NR * NS)) * (NR * NS)
    e = edge_index.shape[1]
    e_pad = ((e + NS * CHUNK - 1) // (NS * CHUNK)) * (NS * CHUNK)

    def halves(x):
        x3 = jnp.concatenate([x.reshape(n, 2, HALF),
                              jnp.zeros((1, 2, HALF), jnp.float32)])
        return x3.transpose(1, 0, 2).reshape(2 * (n + 1), HALF)

    kt = halves(k.reshape(n, HIDDEN))
    qt = halves(q.reshape(n, HIDDEN))
    vt = halves(v.reshape(n, HIDDEN))

    src = edge_index[0].astype(jnp.int32)
    dst = edge_index[1].astype(jnp.int32)
    pad = e_pad - e
    src_pad = jnp.concatenate([src, jnp.full((pad,), n, jnp.int32)])
    dst_pad = jnp.concatenate([dst, jnp.full((pad,), n, jnp.int32)])

    out, _ = _sc_attention(n, n_pad, e_pad, kt, qt, vt, src_pad, dst_pad)
    x = jnp.concatenate([out[:n], out[n_pad:n_pad + n]], axis=1)
    return x.reshape(b, n, HIDDEN)
